# baseline (speedup 1.0000x reference)
import jax
import jax.numpy as jnp
from jax.experimental import pallas as pl
from jax.experimental.pallas import tpu as pltpu

_BN = 2048
_NBW = 3
_NBO = 3
_LOOK = 2


def _make_body(n_full, n_tail):
    def _proj_body(x_ref, w_hbm, b_hbm, o_hbm,
                   w_bufs, b_bufs, o_bufs, wt_buf, bt_buf, ot_buf,
                   w_sems, b_sems, o_sems, t_sems):
        i = pl.program_id(0)
        tail_off = n_full * _BN

        def w_copy(j, slot):
            return pltpu.make_async_copy(
                w_hbm.at[pl.ds(j * _BN, _BN), :], w_bufs.at[slot],
                w_sems.at[slot])

        def b_copy(j, slot):
            return pltpu.make_async_copy(
                b_hbm.at[:, pl.ds(j * _BN, _BN)], b_bufs.at[slot],
                b_sems.at[slot])

        def o_copy(j, slot):
            return pltpu.make_async_copy(
                o_bufs.at[slot], o_hbm.at[:, pl.ds(j * _BN, _BN)],
                o_sems.at[slot])

        @pl.when(i == 0)
        def _():
            for j in range(min(_LOOK + 1, n_full)):
                w_copy(j, j % _NBW).start()
                b_copy(j, j % _NBW).start()
            if n_tail:
                pltpu.make_async_copy(
                    w_hbm.at[pl.ds(tail_off, n_tail), :], wt_buf,
                    t_sems.at[0]).start()
                pltpu.make_async_copy(
                    b_hbm.at[:, pl.ds(tail_off, n_tail)], bt_buf,
                    t_sems.at[1]).start()

        @pl.when((i > 0) & (i + _LOOK < n_full))
        def _():
            j = i + _LOOK
            slot = jax.lax.rem(j, _NBW)
            w_copy(j, slot).start()
            b_copy(j, slot).start()

        x = x_ref[...]

        @pl.when(i < n_full)
        def _():
            slot = jax.lax.rem(i, _NBW)
            w_copy(i, slot).wait()
            b_copy(i, slot).wait()
            oslot = jax.lax.rem(i, _NBO)

            @pl.when(i >= _NBO)
            def _():
                o_copy(i - _NBO, oslot).wait()

            acc = jax.lax.dot_general(
                x, w_bufs[slot], (((1,), (1,)), ((), ())),
                preferred_element_type=jnp.float32,
            )
            o_bufs[oslot] = acc + b_bufs[slot]
            o_copy(i, oslot).start()

        if n_tail:
            @pl.when(i == n_full)
            def _():
                pltpu.make_async_copy(
                    w_hbm.at[pl.ds(tail_off, n_tail), :], wt_buf,
                    t_sems.at[0]).wait()
                pltpu.make_async_copy(
                    b_hbm.at[:, pl.ds(tail_off, n_tail)], bt_buf,
                    t_sems.at[1]).wait()
                acc = jax.lax.dot_general(
                    x, wt_buf[...], (((1,), (1,)), ((), ())),
                    preferred_element_type=jnp.float32,
                )
                ot_buf[...] = acc + bt_buf[...]
                pltpu.make_async_copy(
                    ot_buf, o_hbm.at[:, pl.ds(tail_off, n_tail)],
                    t_sems.at[2]).start()
                pltpu.make_async_copy(
                    ot_buf, o_hbm.at[:, pl.ds(tail_off, n_tail)],
                    t_sems.at[2]).wait()

        last = n_full + (1 if n_tail else 0) - 1

        @pl.when(i == last)
        def _():
            for d in range(min(_NBO, n_full)):
                j = n_full - 1 - d
                o_copy(j, j % _NBO).wait()

    return _proj_body


def kernel(inputs, labels, W, b):
    M, K = inputs.shape
    N = W.shape[0]
    n_full = N // _BN
    n_tail = N - n_full * _BN
    grid = n_full + (1 if n_tail else 0)
    b2 = b.reshape(1, N)
    tail_shapes = [
        pltpu.MemorySpace.VMEM((max(n_tail, 8), K), jnp.float32),
        pltpu.MemorySpace.VMEM((1, max(n_tail, 128)), jnp.float32),
        pltpu.MemorySpace.VMEM((M, max(n_tail, 128)), jnp.float32),
    ]
    logits = pl.pallas_call(
        _make_body(n_full, n_tail),
        grid=(grid,),
        in_specs=[
            pl.BlockSpec((M, K), lambda i: (0, 0)),
            pl.BlockSpec(memory_space=pltpu.MemorySpace.HBM),
            pl.BlockSpec(memory_space=pltpu.MemorySpace.HBM),
        ],
        out_specs=pl.BlockSpec(memory_space=pltpu.MemorySpace.HBM),
        out_shape=jax.ShapeDtypeStruct((M, N), jnp.float32),
        scratch_shapes=[
            pltpu.MemorySpace.VMEM((_NBW, _BN, K), jnp.float32),
            pltpu.MemorySpace.VMEM((_NBW, 1, _BN), jnp.float32),
            pltpu.MemorySpace.VMEM((_NBO, M, _BN), jnp.float32),
        ] + tail_shapes + [
            pltpu.SemaphoreType.DMA((_NBW,)),
            pltpu.SemaphoreType.DMA((_NBW,)),
            pltpu.SemaphoreType.DMA((_NBO,)),
            pltpu.SemaphoreType.DMA((3,)),
        ],
        compiler_params=pltpu.CompilerParams(
            dimension_semantics=("arbitrary",),
        ),
    )(inputs, W, b2)
    return (logits, labels)
